# baseline (device time: 99833 ns/iter reference)
import jax
import jax.numpy as jnp
from jax import lax
from jax.experimental import pallas as pl
from jax.experimental.pallas import tpu as pltpu

N_DEV = 8
B = 2
S_LOC = 128
S = N_DEV * S_LOC
D = 512
H_LOC = 8
DH = 64
SCALE = 0.125

NR = N_DEV // 2
NL = N_DEV - 1 - NR

F32 = jnp.float32
BF16 = jnp.bfloat16


def kernel(x, Wq, Wo, Wk, Wv):
    def body(x_ref, wq_ref, wo_ref, wk_ref, wv_ref, out_ref,
             x16, rbuf, lbuf, q_buf, k_buf, v_buf, pc_buf, sbufR, sbufL,
             rsR, rsL, wqkv16, wo16,
             agr_send, agr_recv, agl_send, agl_recv,
             rsr_send, rsr_recv, rsl_send, rsl_recv):
        my = lax.axis_index("i")
        left = (my - 1) % N_DEV
        right = (my + 1) % N_DEV

        barrier = pltpu.get_barrier_semaphore()
        for nbr in (left, right):
            pl.semaphore_signal(
                barrier, inc=1,
                device_id=(nbr,), device_id_type=pl.DeviceIdType.MESH,
            )
        pl.semaphore_wait(barrier, 2)

        x16[...] = x_ref[...].astype(BF16)
        wqkv16[:, 0:D] = wq_ref[...].astype(BF16)
        wqkv16[:, D:2 * D] = wk_ref[...].astype(BF16)
        wqkv16[:, 2 * D:3 * D] = wv_ref[...].astype(BF16)
        wo16[...] = wo_ref[...].astype(BF16)

        col = lax.broadcasted_iota(jnp.int32, (S, H_LOC * 2 * DH), 1)
        vpat = jnp.where(col % (2 * DH) == DH, 1.0, 0.0).astype(BF16)
        for b in range(B):
            v_buf[b] = vpat

        def qkv_chunk(chunk, seq_off):
            xc = chunk.reshape(B * S_LOC, D)
            r16 = jnp.dot(
                xc, wqkv16[...], preferred_element_type=F32).astype(BF16)
            for b in range(B):
                rb = r16[b * S_LOC:(b + 1) * S_LOC]
                q_buf[b, pl.ds(seq_off, S_LOC), :] = rb[:, 0:D]
                k_buf[b, pl.ds(seq_off, S_LOC), :] = rb[:, D:2 * D]
                for h in range(H_LOC):
                    v_buf[b, pl.ds(seq_off, S_LOC),
                          h * 2 * DH:h * 2 * DH + DH] = (
                        rb[:, 2 * D + h * DH:2 * D + (h + 1) * DH])

        def agr_rdma(h):
            return pltpu.make_async_remote_copy(
                src_ref=x16 if h == 0 else rbuf.at[h - 1],
                dst_ref=rbuf.at[h],
                send_sem=agr_send.at[h],
                recv_sem=agr_recv.at[h],
                device_id=(right,),
                device_id_type=pl.DeviceIdType.MESH,
            )

        def agl_rdma(h):
            return pltpu.make_async_remote_copy(
                src_ref=x16 if h == 0 else lbuf.at[h - 1],
                dst_ref=lbuf.at[h],
                send_sem=agl_send.at[h],
                recv_sem=agl_recv.at[h],
                device_id=(left,),
                device_id_type=pl.DeviceIdType.MESH,
            )

        agr_rdma(0).start()
        agl_rdma(0).start()
        qkv_chunk(x16[...], my * S_LOC)
        for h in range(NR):
            agr_rdma(h).wait_recv()
            if h + 1 < NR:
                agr_rdma(h + 1).start()
            if h < NL:
                agl_rdma(h).wait_recv()
                if h + 1 < NL:
                    agl_rdma(h + 1).start()
            qkv_chunk(rbuf[h], ((my - h - 1) % N_DEV) * S_LOC)
            if h < NL:
                qkv_chunk(lbuf[h], ((my + h + 1) % N_DEV) * S_LOC)
        for h in range(NR):
            agr_rdma(h).wait_send()
        for h in range(NL):
            agl_rdma(h).wait_send()

        def p_chunk_into(c, dst):
            c_off = (c % N_DEV) * S_LOC
            for b in range(B):
                def hpbody(hp, acc):
                    off = hp * 2 * DH
                    q2 = q_buf[b, pl.ds(c_off, S_LOC), pl.ds(off, 2 * DH)]
                    k2 = k_buf[b, :, pl.ds(off, 2 * DH)]
                    v2 = v_buf[b, :, pl.ds(2 * off, 4 * DH)]
                    w2 = wo16[pl.ds(off, 2 * DH), :]
                    for j in range(2):
                        qh = q2[:, j * DH:(j + 1) * DH]
                        kh = k2[:, j * DH:(j + 1) * DH]
                        vh = v2[:, j * 2 * DH:(j + 1) * 2 * DH]
                        wh = w2[j * DH:(j + 1) * DH, :]
                        s = lax.dot_general(
                            qh, kh, (((1,), (1,)), ((), ())),
                            preferred_element_type=F32) * SCALE
                        p = jnp.exp(s).astype(BF16)
                        ov = jnp.dot(p, vh, preferred_element_type=F32)
                        o = ov[:, 0:DH] / ov[:, DH:DH + 1]
                        acc = acc + jnp.dot(
                            o.astype(BF16), wh, preferred_element_type=F32)
                    return acc
                dst[b] = lax.fori_loop(
                    0, H_LOC // 2, hpbody,
                    jnp.zeros((S_LOC, D), F32)).astype(dst.dtype)

        def rsr_rdma(s, src):
            return pltpu.make_async_remote_copy(
                src_ref=src,
                dst_ref=rsR.at[s],
                send_sem=rsr_send.at[s],
                recv_sem=rsr_recv.at[s],
                device_id=(right,),
                device_id_type=pl.DeviceIdType.MESH,
            )

        def rsl_rdma(s, src):
            return pltpu.make_async_remote_copy(
                src_ref=src,
                dst_ref=rsL.at[s],
                send_sem=rsl_send.at[s],
                recv_sem=rsl_recv.at[s],
                device_id=(left,),
                device_id_type=pl.DeviceIdType.MESH,
            )

        p_chunk_into(my + 3, sbufR)
        rsr_rdma(0, sbufR).start()
        p_chunk_into(my - 4, sbufL)
        rsl_rdma(0, sbufL).start()
        for s in range(1, 4):
            if s < 3:
                p_chunk_into(my + 3 - s, pc_buf)
                rsr_rdma(s - 1, sbufR).wait_recv()
                rsR[s - 1] = (rsR[s - 1].astype(F32)
                              + pc_buf[...]).astype(BF16)
                rsr_rdma(s, rsR.at[s - 1]).start()
            p_chunk_into(my - 4 + s, pc_buf)
            rsl_rdma(s - 1, sbufL).wait_recv()
            rsL[s - 1] = (rsL[s - 1].astype(F32)
                          + pc_buf[...]).astype(BF16)
            rsl_rdma(s, rsL.at[s - 1]).start()
        p_chunk_into(my, pc_buf)
        rsr_rdma(2, sbufR).wait_recv()
        rsl_rdma(3, sbufL).wait_recv()
        out_ref[...] = (pc_buf[...] + rsR[2].astype(F32)
                        + rsL[3].astype(F32))
        for s in range(3):
            rsr_rdma(s, sbufR).wait_send()
        for s in range(4):
            rsl_rdma(s, sbufL).wait_send()

    return pl.pallas_call(
        body,
        out_shape=jax.ShapeDtypeStruct((B, S_LOC, D), F32),
        in_specs=[pl.BlockSpec(memory_space=pltpu.VMEM)] * 5,
        out_specs=pl.BlockSpec(memory_space=pltpu.VMEM),
        scratch_shapes=[
            pltpu.VMEM((B, S_LOC, D), BF16),
            pltpu.VMEM((NR, B, S_LOC, D), BF16),
            pltpu.VMEM((NL, B, S_LOC, D), BF16),
            pltpu.VMEM((B, S, D), BF16),
            pltpu.VMEM((B, S, D), BF16),
            pltpu.VMEM((B, S, H_LOC * 2 * DH), BF16),
            pltpu.VMEM((B, S_LOC, D), F32),
            pltpu.VMEM((B, S_LOC, D), BF16),
            pltpu.VMEM((B, S_LOC, D), BF16),
            pltpu.VMEM((3, B, S_LOC, D), BF16),
            pltpu.VMEM((4, B, S_LOC, D), BF16),
            pltpu.VMEM((D, 3 * D), BF16),
            pltpu.VMEM((D, D), BF16),
            pltpu.SemaphoreType.DMA((NR,)),
            pltpu.SemaphoreType.DMA((NR,)),
            pltpu.SemaphoreType.DMA((NL,)),
            pltpu.SemaphoreType.DMA((NL,)),
            pltpu.SemaphoreType.DMA((3,)),
            pltpu.SemaphoreType.DMA((3,)),
            pltpu.SemaphoreType.DMA((4,)),
            pltpu.SemaphoreType.DMA((4,)),
        ],
        compiler_params=pltpu.CompilerParams(
            collective_id=0,
            vmem_limit_bytes=100 * 1024 * 1024,
        ),
    )(x, Wq, Wo, Wk, Wv)


# device time: 74531 ns/iter; 1.3395x vs baseline; 1.3395x over previous
import jax
import jax.numpy as jnp
from jax import lax
from jax.experimental import pallas as pl
from jax.experimental.pallas import tpu as pltpu

N_DEV = 8
B = 2
S_LOC = 128
S = N_DEV * S_LOC
D = 512
H_LOC = 8
DH = 64
SCALE = 0.125

NR = N_DEV // 2
NL = N_DEV - 1 - NR

F32 = jnp.float32
BF16 = jnp.bfloat16


def kernel(x, Wq, Wo, Wk, Wv):
    def body(x_ref, wq_ref, wo_ref, wk_ref, wv_ref, out_ref,
             x16, rbuf, lbuf, q_buf, k_buf, v_buf, pc2a, pc2b,
             rsR, rsL, wqkv16, wo16,
             agr_send, agr_recv, agl_send, agl_recv,
             rsr_send, rsr_recv, rsl_send, rsl_recv):
        my = lax.axis_index("i")
        left = (my - 1) % N_DEV
        right = (my + 1) % N_DEV

        barrier = pltpu.get_barrier_semaphore()
        for nbr in (left, right):
            pl.semaphore_signal(
                barrier, inc=1,
                device_id=(nbr,), device_id_type=pl.DeviceIdType.MESH,
            )
        pl.semaphore_wait(barrier, 2)

        x16[...] = x_ref[...].astype(BF16)
        wqkv16[:, 0:D] = wq_ref[...].astype(BF16)
        wqkv16[:, D:2 * D] = wk_ref[...].astype(BF16)
        wqkv16[:, 2 * D:3 * D] = wv_ref[...].astype(BF16)
        wo16[...] = wo_ref[...].astype(BF16)

        def qkv_chunk(chunk, seq_off):
            xc = chunk.reshape(B * S_LOC, D)
            r16 = jnp.dot(
                xc, wqkv16[...], preferred_element_type=F32).astype(BF16)
            for b in range(B):
                rb = r16[b * S_LOC:(b + 1) * S_LOC]
                q_buf[b, pl.ds(seq_off, S_LOC), :] = rb[:, 0:D]
                k_buf[b, pl.ds(seq_off, S_LOC), :] = rb[:, D:2 * D]
                v_buf[b, pl.ds(seq_off, S_LOC), :] = rb[:, 2 * D:3 * D]

                @pl.when(seq_off == 0)
                def _(b=b, rb=rb):
                    q_buf[b, S:S + S_LOC, :] = rb[:, 0:D]

        def agr_rdma(h):
            return pltpu.make_async_remote_copy(
                src_ref=x16 if h == 0 else rbuf.at[h - 1],
                dst_ref=rbuf.at[h],
                send_sem=agr_send.at[h],
                recv_sem=agr_recv.at[h],
                device_id=(right,),
                device_id_type=pl.DeviceIdType.MESH,
            )

        def agl_rdma(h):
            return pltpu.make_async_remote_copy(
                src_ref=x16 if h == 0 else lbuf.at[h - 1],
                dst_ref=lbuf.at[h],
                send_sem=agl_send.at[h],
                recv_sem=agl_recv.at[h],
                device_id=(left,),
                device_id_type=pl.DeviceIdType.MESH,
            )

        agr_rdma(0).start()
        agl_rdma(0).start()
        qkv_chunk(x16[...], my * S_LOC)
        for h in range(NR):
            agr_rdma(h).wait_recv()
            if h + 1 < NR:
                agr_rdma(h + 1).start()
            if h < NL:
                agl_rdma(h).wait_recv()
                if h + 1 < NL:
                    agl_rdma(h + 1).start()
            qkv_chunk(rbuf[h], ((my - h - 1) % N_DEV) * S_LOC)
            if h < NL:
                qkv_chunk(lbuf[h], ((my + h + 1) % N_DEV) * S_LOC)
        for h in range(NR):
            agr_rdma(h).wait_send()
        for h in range(NL):
            agl_rdma(h).wait_send()

        def p2_chunk_into(c, dst):
            c_off = (c % N_DEV) * S_LOC
            for b in range(B):
                def hpbody(hp, acc):
                    off = hp * 2 * DH
                    q2 = q_buf[b, pl.ds(c_off, 2 * S_LOC), pl.ds(off, 2 * DH)]
                    k2 = k_buf[b, :, pl.ds(off, 2 * DH)]
                    v2 = v_buf[b, :, pl.ds(off, 2 * DH)]
                    w2 = wo16[pl.ds(off, 2 * DH), :]
                    for j in range(2):
                        qh = q2[:, j * DH:(j + 1) * DH]
                        kh = k2[:, j * DH:(j + 1) * DH]
                        vh = v2[:, j * DH:(j + 1) * DH]
                        wh = w2[j * DH:(j + 1) * DH, :]
                        s = lax.dot_general(
                            qh, kh, (((1,), (1,)), ((), ())),
                            preferred_element_type=F32) * SCALE
                        p = jnp.exp(s)
                        l = jnp.sum(p, axis=1, keepdims=True)
                        o = jnp.dot(p.astype(BF16), vh,
                                    preferred_element_type=F32) / l
                        acc = acc + jnp.dot(
                            o.astype(BF16), wh, preferred_element_type=F32)
                    return acc
                dst[b] = lax.fori_loop(
                    0, H_LOC // 2, hpbody,
                    jnp.zeros((2 * S_LOC, D), F32)).astype(dst.dtype)

        def rsr_rdma(s, src):
            return pltpu.make_async_remote_copy(
                src_ref=src,
                dst_ref=rsR.at[s],
                send_sem=rsr_send.at[s],
                recv_sem=rsr_recv.at[s],
                device_id=(right,),
                device_id_type=pl.DeviceIdType.MESH,
            )

        def rsl_rdma(s, src):
            return pltpu.make_async_remote_copy(
                src_ref=src,
                dst_ref=rsL.at[s],
                send_sem=rsl_send.at[s],
                recv_sem=rsl_recv.at[s],
                device_id=(left,),
                device_id_type=pl.DeviceIdType.MESH,
            )

        a_lo, a_hi = pc2a.at[:, 0:S_LOC, :], pc2a.at[:, S_LOC:2 * S_LOC, :]
        b_lo, b_hi = pc2b.at[:, 0:S_LOC, :], pc2b.at[:, S_LOC:2 * S_LOC, :]
        dummy = a_lo

        p2_chunk_into(my + 3, pc2a)
        rsr_rdma(0, a_lo).start()
        rsl_rdma(0, a_hi).start()

        p2_chunk_into(my + 1, pc2b)
        rsr_rdma(0, dummy).wait_recv()
        rsR[0] = (rsR[0].astype(F32) + b_hi[...].astype(F32)).astype(BF16)
        rsr_rdma(1, rsR.at[0]).start()

        rsr_rdma(0, a_lo).wait_send()
        rsl_rdma(0, a_hi).wait_send()
        p2_chunk_into(my - 3, pc2a)
        rsl_rdma(0, dummy).wait_recv()
        rsL[0] = (rsL[0].astype(F32) + a_lo[...].astype(F32)).astype(BF16)
        rsl_rdma(1, rsL.at[0]).start()

        rsr_rdma(1, dummy).wait_recv()
        rsR[1] = (rsR[1].astype(F32) + b_lo[...].astype(F32)).astype(BF16)
        rsr_rdma(2, rsR.at[1]).start()

        rsl_rdma(1, dummy).wait_recv()
        rsL[1] = (rsL[1].astype(F32) + a_hi[...].astype(F32)).astype(BF16)
        rsl_rdma(2, rsL.at[1]).start()

        p2_chunk_into(my - 1, pc2b)
        rsl_rdma(2, dummy).wait_recv()
        rsL[2] = (rsL[2].astype(F32) + b_lo[...].astype(F32)).astype(BF16)
        rsl_rdma(3, rsL.at[2]).start()

        rsr_rdma(2, dummy).wait_recv()
        rsl_rdma(3, dummy).wait_recv()
        out_ref[...] = (b_hi[...].astype(F32) + rsR[2].astype(F32)
                        + rsL[3].astype(F32))
        for s in range(1, 3):
            rsr_rdma(s, dummy).wait_send()
        for s in range(1, 4):
            rsl_rdma(s, dummy).wait_send()

    return pl.pallas_call(
        body,
        out_shape=jax.ShapeDtypeStruct((B, S_LOC, D), F32),
        in_specs=[pl.BlockSpec(memory_space=pltpu.VMEM)] * 5,
        out_specs=pl.BlockSpec(memory_space=pltpu.VMEM),
        scratch_shapes=[
            pltpu.VMEM((B, S_LOC, D), BF16),
            pltpu.VMEM((NR, B, S_LOC, D), BF16),
            pltpu.VMEM((NL, B, S_LOC, D), BF16),
            pltpu.VMEM((B, S + S_LOC, D), BF16),
            pltpu.VMEM((B, S, D), BF16),
            pltpu.VMEM((B, S, D), BF16),
            pltpu.VMEM((B, 2 * S_LOC, D), BF16),
            pltpu.VMEM((B, 2 * S_LOC, D), BF16),
            pltpu.VMEM((3, B, S_LOC, D), BF16),
            pltpu.VMEM((4, B, S_LOC, D), BF16),
            pltpu.VMEM((D, 3 * D), BF16),
            pltpu.VMEM((D, D), BF16),
            pltpu.SemaphoreType.DMA((NR,)),
            pltpu.SemaphoreType.DMA((NR,)),
            pltpu.SemaphoreType.DMA((NL,)),
            pltpu.SemaphoreType.DMA((NL,)),
            pltpu.SemaphoreType.DMA((3,)),
            pltpu.SemaphoreType.DMA((3,)),
            pltpu.SemaphoreType.DMA((4,)),
            pltpu.SemaphoreType.DMA((4,)),
        ],
        compiler_params=pltpu.CompilerParams(
            collective_id=0,
            vmem_limit_bytes=100 * 1024 * 1024,
        ),
    )(x, Wq, Wo, Wk, Wv)
